# baseline (device time: 65511 ns/iter reference)
import jax
import jax.numpy as jnp
from jax import lax
from jax.experimental import pallas as pl
from jax.experimental.pallas import tpu as pltpu

N_DEV = 4
B = 2
SQ = 256
SKV_PER = 256
H_PER = 4
DH = 64
D_MODEL = 512
BLK = 64


def kernel(x, Wq, K_ext, V_ext, Wo):
    def body(x_ref, wq_ref, k_ref, v_ref, wo_ref, out_ref,
             k_loc, v_loc, kbuf, vbuf, q_buf, s_buf, w_buf, ctx_buf, o_parts,
             ksend, krecv, vsend, vrecv, osend, orecv):
        my = lax.axis_index("i")

        for g in range(N_DEV):
            k_loc[g] = k_ref[:, :, H_PER * g:H_PER * (g + 1), :].astype(jnp.bfloat16)
            v_loc[g] = v_ref[:, :, H_PER * g:H_PER * (g + 1), :].astype(jnp.bfloat16)

        bsem = pltpu.get_barrier_semaphore()
        for k in range(1, N_DEV):
            pl.semaphore_signal(
                bsem, inc=1,
                device_id=((my + k) % N_DEV,),
                device_id_type=pl.DeviceIdType.MESH,
            )
        pl.semaphore_wait(bsem, N_DEV - 1)

        kv_rdmas = []
        for k in range(1, N_DEV):
            d = (my + k) % N_DEV
            for loc, buf, ssem, rsem in ((k_loc, kbuf, ksend, krecv),
                                         (v_loc, vbuf, vsend, vrecv)):
                rdma = pltpu.make_async_remote_copy(
                    src_ref=loc.at[d],
                    dst_ref=buf.at[k],
                    send_sem=ssem.at[k],
                    recv_sem=rsem.at[k],
                    device_id=(d,),
                    device_id_type=pl.DeviceIdType.MESH,
                )
                rdma.start()
                kv_rdmas.append(rdma)

        kbuf[0] = k_loc[my]
        vbuf[0] = v_loc[my]

        for b in range(B):
            xb = x_ref[b].astype(jnp.bfloat16)
            for h in range(H_PER):
                q_buf[b, h] = lax.dot_general(
                    xb, wq_ref[:, DH * h:DH * (h + 1)].astype(jnp.bfloat16),
                    (((1,), (0,)), ((), ())),
                    preferred_element_type=jnp.float32,
                ).astype(jnp.bfloat16)

        for rdma in kv_rdmas:
            rdma.wait_recv()

        qb_i = lax.broadcasted_iota(jnp.int32, (SQ, SKV_PER), 0) // BLK
        kb_i = lax.broadcasted_iota(jnp.int32, (SQ, SKV_PER), 1) // BLK
        for b in range(B):
            for h in range(H_PER):
                q_bh = q_buf[b, h]
                for k in range(N_DEV):
                    src = (my - k) % N_DEV
                    sblk = lax.dot_general(
                        q_bh, kbuf[k, b, :, h, :],
                        (((1,), (1,)), ((), ())),
                        preferred_element_type=jnp.float32,
                    ) * 0.125
                    kb = src * (SKV_PER // BLK) + kb_i
                    m = (qb_i == kb) | (kb == 0) | ((qb_i + kb) % 3 == 0)
                    s_buf[:, SKV_PER * k:SKV_PER * (k + 1)] = jnp.where(m, sblk, -1e9)
                s = s_buf[...]
                mx = jnp.max(s, axis=1, keepdims=True)
                w = jnp.exp(s - mx)
                w_buf[...] = (w / jnp.sum(w, axis=1, keepdims=True)).astype(jnp.bfloat16)
                acc = jnp.zeros((SQ, DH), jnp.float32)
                for k in range(N_DEV):
                    acc = acc + lax.dot_general(
                        w_buf[:, SKV_PER * k:SKV_PER * (k + 1)],
                        vbuf[k, b, :, h, :],
                        (((1,), (0,)), ((), ())),
                        preferred_element_type=jnp.float32,
                    )
                ctx_buf[b, :, h, :] = acc

        wo_b = wo_ref[...].astype(jnp.bfloat16)
        for b in range(B):
            cb = ctx_buf[b].reshape(SQ, H_PER * DH).astype(jnp.bfloat16)
            o_parts[0, b] = lax.dot_general(
                cb, wo_b, (((1,), (0,)), ((), ())),
                preferred_element_type=jnp.float32,
            ).astype(jnp.bfloat16)

        o_rdmas = []
        for k in range(1, N_DEV):
            d = (my + k) % N_DEV
            rdma = pltpu.make_async_remote_copy(
                src_ref=o_parts.at[0],
                dst_ref=o_parts.at[k],
                send_sem=osend.at[k],
                recv_sem=orecv.at[k],
                device_id=(d,),
                device_id_type=pl.DeviceIdType.MESH,
            )
            rdma.start()
            o_rdmas.append(rdma)
        for rdma in o_rdmas:
            rdma.wait_recv()

        out_ref[...] = (o_parts[0].astype(jnp.float32)
                        + o_parts[1].astype(jnp.float32)
                        + o_parts[2].astype(jnp.float32)
                        + o_parts[3].astype(jnp.float32))

        for rdma in kv_rdmas + o_rdmas:
            rdma.wait_send()

    return pl.pallas_call(
        body,
        out_shape=jax.ShapeDtypeStruct((B, SQ, D_MODEL), jnp.float32),
        in_specs=[pl.BlockSpec(memory_space=pltpu.VMEM)] * 5,
        out_specs=pl.BlockSpec(memory_space=pltpu.VMEM),
        scratch_shapes=[
            pltpu.VMEM((N_DEV, B, SKV_PER, H_PER, DH), jnp.bfloat16),
            pltpu.VMEM((N_DEV, B, SKV_PER, H_PER, DH), jnp.bfloat16),
            pltpu.VMEM((N_DEV, B, SKV_PER, H_PER, DH), jnp.bfloat16),
            pltpu.VMEM((N_DEV, B, SKV_PER, H_PER, DH), jnp.bfloat16),
            pltpu.VMEM((B, H_PER, SQ, DH), jnp.bfloat16),
            pltpu.VMEM((SQ, N_DEV * SKV_PER), jnp.float32),
            pltpu.VMEM((SQ, N_DEV * SKV_PER), jnp.bfloat16),
            pltpu.VMEM((B, SQ, H_PER, DH), jnp.float32),
            pltpu.VMEM((N_DEV, B, SQ, D_MODEL), jnp.bfloat16),
            pltpu.SemaphoreType.DMA((N_DEV,)),
            pltpu.SemaphoreType.DMA((N_DEV,)),
            pltpu.SemaphoreType.DMA((N_DEV,)),
            pltpu.SemaphoreType.DMA((N_DEV,)),
            pltpu.SemaphoreType.DMA((N_DEV,)),
            pltpu.SemaphoreType.DMA((N_DEV,)),
        ],
        compiler_params=pltpu.CompilerParams(collective_id=0),
    )(x, Wq, K_ext, V_ext, Wo)


# device time: 61024 ns/iter; 1.0735x vs baseline; 1.0735x over previous
import jax
import jax.numpy as jnp
from jax import lax
from jax.experimental import pallas as pl
from jax.experimental.pallas import tpu as pltpu

N_DEV = 4
B = 2
SQ = 256
SKV_PER = 256
H_PER = 4
DH = 64
D_MODEL = 512
BLK = 64
NBLK_PER = SKV_PER // BLK


def kernel(x, Wq, K_ext, V_ext, Wo):
    def body(x_ref, wq_ref, k_ref, v_ref, wo_ref, out_ref,
             k_loc, v_loc, kbuf, vbuf, q_buf, ctx_buf, o_parts,
             ksend, krecv, vsend, vrecv, osend, orecv):
        my = lax.axis_index("i")

        for g in range(N_DEV):
            for b in range(B):
                for h in range(H_PER):
                    k_loc[g, b, h] = k_ref[b, :, H_PER * g + h, :].astype(jnp.bfloat16)
                    v_loc[g, b, h] = v_ref[b, :, H_PER * g + h, :].astype(jnp.bfloat16)

        bsem = pltpu.get_barrier_semaphore()
        for k in range(1, N_DEV):
            pl.semaphore_signal(
                bsem, inc=1,
                device_id=((my + k) % N_DEV,),
                device_id_type=pl.DeviceIdType.MESH,
            )
        pl.semaphore_wait(bsem, N_DEV - 1)

        kv_rdmas = []
        for k in range(1, N_DEV):
            d = (my + k) % N_DEV
            for loc, buf, ssem, rsem in ((k_loc, kbuf, ksend, krecv),
                                         (v_loc, vbuf, vsend, vrecv)):
                rdma = pltpu.make_async_remote_copy(
                    src_ref=loc.at[d],
                    dst_ref=buf.at[:, :, k],
                    send_sem=ssem.at[k],
                    recv_sem=rsem.at[k],
                    device_id=(d,),
                    device_id_type=pl.DeviceIdType.MESH,
                )
                rdma.start()
                kv_rdmas.append(rdma)

        kbuf[:, :, 0] = k_loc[my]
        vbuf[:, :, 0] = v_loc[my]

        for b in range(B):
            q_buf[b] = lax.dot_general(
                x_ref[b].astype(jnp.bfloat16), wq_ref[...].astype(jnp.bfloat16),
                (((1,), (0,)), ((), ())),
                preferred_element_type=jnp.float32,
            ).astype(jnp.bfloat16)

        for rdma in kv_rdmas:
            rdma.wait_recv()

        qb = lax.broadcasted_iota(jnp.int32, (SQ, N_DEV * SKV_PER), 0) // BLK
        col = lax.broadcasted_iota(jnp.int32, (SQ, N_DEV * SKV_PER), 1)
        src = (my - col // SKV_PER) % N_DEV
        kb = src * NBLK_PER + (col % SKV_PER) // BLK
        mask = (qb == kb) | (kb == 0) | ((qb + kb) % 3 == 0)

        wo_b = wo_ref[...].astype(jnp.bfloat16)
        o_rdmas = []
        for b in range(B):
            for h in range(H_PER):
                k_bh = kbuf[b, h].reshape(N_DEV * SKV_PER, DH)
                s = lax.dot_general(
                    q_buf[b, :, DH * h:DH * (h + 1)], k_bh,
                    (((1,), (1,)), ((), ())),
                    preferred_element_type=jnp.float32,
                ) * 0.125
                w = jnp.exp(jnp.where(mask, s, -1e9))
                w = (w / jnp.sum(w, axis=1, keepdims=True)).astype(jnp.bfloat16)
                ctx_buf[b, :, DH * h:DH * (h + 1)] = lax.dot_general(
                    w, vbuf[b, h].reshape(N_DEV * SKV_PER, DH),
                    (((1,), (0,)), ((), ())),
                    preferred_element_type=jnp.float32,
                )
            o_parts[0, b] = lax.dot_general(
                ctx_buf[b].astype(jnp.bfloat16), wo_b,
                (((1,), (0,)), ((), ())),
                preferred_element_type=jnp.float32,
            ).astype(jnp.bfloat16)
            for k in range(1, N_DEV):
                d = (my + k) % N_DEV
                rdma = pltpu.make_async_remote_copy(
                    src_ref=o_parts.at[0, b],
                    dst_ref=o_parts.at[k, b],
                    send_sem=osend.at[k, b],
                    recv_sem=orecv.at[k, b],
                    device_id=(d,),
                    device_id_type=pl.DeviceIdType.MESH,
                )
                rdma.start()
                o_rdmas.append(rdma)

        for rdma in o_rdmas:
            rdma.wait_recv()

        out_ref[...] = (o_parts[0].astype(jnp.float32)
                        + o_parts[1].astype(jnp.float32)
                        + o_parts[2].astype(jnp.float32)
                        + o_parts[3].astype(jnp.float32))

        for rdma in kv_rdmas + o_rdmas:
            rdma.wait_send()

    return pl.pallas_call(
        body,
        out_shape=jax.ShapeDtypeStruct((B, SQ, D_MODEL), jnp.float32),
        in_specs=[pl.BlockSpec(memory_space=pltpu.VMEM)] * 5,
        out_specs=pl.BlockSpec(memory_space=pltpu.VMEM),
        scratch_shapes=[
            pltpu.VMEM((N_DEV, B, H_PER, SKV_PER, DH), jnp.bfloat16),
            pltpu.VMEM((N_DEV, B, H_PER, SKV_PER, DH), jnp.bfloat16),
            pltpu.VMEM((B, H_PER, N_DEV, SKV_PER, DH), jnp.bfloat16),
            pltpu.VMEM((B, H_PER, N_DEV, SKV_PER, DH), jnp.bfloat16),
            pltpu.VMEM((B, SQ, H_PER * DH), jnp.bfloat16),
            pltpu.VMEM((B, SQ, H_PER * DH), jnp.float32),
            pltpu.VMEM((N_DEV, B, SQ, D_MODEL), jnp.bfloat16),
            pltpu.SemaphoreType.DMA((N_DEV,)),
            pltpu.SemaphoreType.DMA((N_DEV,)),
            pltpu.SemaphoreType.DMA((N_DEV,)),
            pltpu.SemaphoreType.DMA((N_DEV,)),
            pltpu.SemaphoreType.DMA((N_DEV, B)),
            pltpu.SemaphoreType.DMA((N_DEV, B)),
        ],
        compiler_params=pltpu.CompilerParams(collective_id=0),
    )(x, Wq, K_ext, V_ext, Wo)
